# Initial kernel scaffold; baseline (speedup 1.0000x reference)
import jax
import jax.numpy as jnp
from jax.experimental import pallas as pl


def kernel(x, edge_index, edge_attr, W_g0, b_g0, Wih0, Whh0, bih0, bhh0, W_g1, b_g1, Wih1, Whh1, bih1, bhh1, W_out, b_out):
    raise NotImplementedError("write your pallas kernel here")


# speedup vs baseline: 4.2467x; 4.2467x over previous
import functools

import jax
import jax.numpy as jnp
from jax import lax
from jax.experimental import pallas as pl
from jax.experimental.pallas import tpu as pltpu
from jax.experimental.pallas import tpu_sc as plsc

F32 = jnp.float32

N_CORES = 2
N_SUBCORES = 16
NW = N_CORES * N_SUBCORES
CHUNK = 128
ROW_BLOCK = 1024
H = 128


def _sigmoid(x):
    return 1.0 / (1.0 + jnp.exp(-x))


@functools.lru_cache(maxsize=None)
def _make_sc_agg(n_pad, n_chunks):
    epw = n_chunks * CHUNK
    rows_per_tile = n_pad // N_SUBCORES
    out_sd = jax.ShapeDtypeStruct((n_pad, H), F32)
    mesh = plsc.VectorSubcoreMesh(core_axis_name="c", subcore_axis_name="s")

    @functools.partial(
        pl.kernel,
        mesh=mesh,
        out_type=(out_sd, out_sd),
        scratch_types=[
            pltpu.VMEM((epw,), jnp.int32),
            pltpu.VMEM((n_chunks, CHUNK), jnp.int32),
            pltpu.VMEM((CHUNK, H), F32),
            pltpu.VMEM_SHARED((n_pad, H), F32),
            pltpu.SemaphoreType.DMA,
        ],
    )
    def agg(y_hbm, zeros_hbm, src_hbm, dst_hbm, s0_hbm, s1_hbm,
            src_v, dst_v, rows_v, acc, sem):
        cid = lax.axis_index("c")
        sid = lax.axis_index("s")
        wid = sid * N_CORES + cid
        rows = pl.ds(sid * rows_per_tile, rows_per_tile)

        @pl.when(cid == 0)
        def _():
            pltpu.sync_copy(y_hbm.at[rows], acc.at[rows])

        @pl.when(cid != 0)
        def _():
            pltpu.sync_copy(zeros_hbm.at[rows], acc.at[rows])

        pltpu.sync_copy(src_hbm.at[pl.ds(wid * epw, epw)], src_v)
        pltpu.sync_copy(dst_hbm.at[pl.ds(wid * n_chunks, n_chunks)], dst_v)
        plsc.subcore_barrier()

        def body(ci, carry):
            idx = src_v.at[pl.ds(ci * CHUNK, CHUNK)]
            pltpu.async_copy(y_hbm.at[idx], rows_v, sem).wait()
            pltpu.sync_copy(rows_v, acc.at[dst_v.at[ci]], add=True)
            return carry

        lax.fori_loop(0, n_chunks, body, 0)
        plsc.subcore_barrier()

        @pl.when(cid == 0)
        def _():
            pltpu.sync_copy(acc.at[rows], s0_hbm.at[rows])

        @pl.when(cid != 0)
        def _():
            pltpu.sync_copy(acc.at[rows], s1_hbm.at[rows])

    return agg


def _tc_call(body, grid, in_specs, out_specs, out_shape):
    return pl.pallas_call(body, grid=grid, in_specs=in_specs,
                          out_specs=out_specs, out_shape=out_shape)


def _rows(b, c):
    return pl.BlockSpec((b, c), lambda i: (i, 0))


def _full(shape):
    return pl.BlockSpec(shape, lambda i: tuple(0 for _ in shape))


def _tca_body(d0, d1, x0, wg0, dinv_o, y0_o):
    dinv = lax.rsqrt(d0[:, :1] + d1[:, :1])
    dinv_o[...] = dinv
    y0_o[...] = dinv * jnp.dot(x0[...], wg0[...], preferred_element_type=F32)


def _lstm_step(s0, s1, dinv, bg, inp_a, wa, wg, h, whh, bsum, c):
    g = _sigmoid(dinv * (s0 + s1) + bg)
    gates = (jnp.dot(inp_a, wa, preferred_element_type=F32)
             + jnp.dot(g, wg, preferred_element_type=F32)
             + jnp.dot(h, whh, preferred_element_type=F32) + bsum)
    ig = _sigmoid(gates[:, 0:H])
    fg = _sigmoid(gates[:, H:2 * H])
    gg = jnp.tanh(gates[:, 2 * H:3 * H])
    og = _sigmoid(gates[:, 3 * H:4 * H])
    c2 = fg * c + ig * gg
    h2 = og * jnp.tanh(c2)
    return h2, c2


def _tc1_body(s0, s1, xt, h0, c0, dinv, bg0, wx, wgg, whh, bsum, wg1,
              h0n, c0n, y1n):
    dv = dinv[...]
    h2, c2 = _lstm_step(s0[...], s1[...], dv, bg0[...], xt[...], wx[...],
                        wgg[...], h0[...], whh[...], bsum[...], c0[...])
    h0n[...] = h2
    c0n[...] = c2
    y1n[...] = dv * jnp.dot(h2, wg1[...], preferred_element_type=F32)


def _tc2_body(s0, s1, h0, h1, c1, xn, dinv, bg1, wh, wgg, whh, bsum, wg0,
              h1n, c1n, y0n):
    dv = dinv[...]
    h2, c2 = _lstm_step(s0[...], s1[...], dv, bg1[...], h0[...], wh[...],
                        wgg[...], h1[...], whh[...], bsum[...], c1[...])
    h1n[...] = h2
    c1n[...] = c2
    y0n[...] = dv * jnp.dot(xn[...], wg0[...], preferred_element_type=F32)


def _tcf_body(h1, wout, bout, out):
    out[...] = jnp.dot(h1[...], wout[...], preferred_element_type=F32) + bout[...]


def kernel(x, edge_index, edge_attr,
           W_g0, b_g0, Wih0, Whh0, bih0, bhh0,
           W_g1, b_g1, Wih1, Whh1, bih1, bhh1,
           W_out, b_out):
    n, f_in, t_steps = x.shape
    n_pad = -(-n // ROW_BLOCK) * ROW_BLOCK
    e = edge_index.shape[1]
    n_chunks = -(-e // (NW * CHUNK))
    n_chunks = -(-n_chunks // 8) * 8
    e_pad = NW * n_chunks * CHUNK
    grid = n_pad // ROW_BLOCK

    src = edge_index[0].astype(jnp.int32)
    dst = edge_index[1].astype(jnp.int32)
    srcp = jnp.concatenate([src, jnp.zeros((e_pad - e,), jnp.int32)])
    dstp = jnp.concatenate([dst, jnp.full((e_pad - e,), n_pad - 1, jnp.int32)])
    dst2d = dstp.reshape(NW * n_chunks, CHUNK)

    zeros = jnp.zeros((n_pad, H), F32)
    ones = jnp.ones((n_pad, H), F32)
    xp = jnp.pad(x, ((0, n_pad - n), (0, 0), (0, 0)))
    xts = jnp.transpose(xp, (2, 0, 1))

    w0t = Wih0.T
    wx0, wg0g = w0t[:f_in], w0t[f_in:]
    whh0t = Whh0.T
    bsum0 = (bih0 + bhh0).reshape(1, 4 * H)
    w1t = Wih1.T
    wh1, wg1g = w1t[:H], w1t[H:]
    whh1t = Whh1.T
    bsum1 = (bih1 + bhh1).reshape(1, 4 * H)
    bg0 = b_g0.reshape(1, H)
    bg1 = b_g1.reshape(1, H)
    boutr = b_out.reshape(1, -1)

    agg = _make_sc_agg(n_pad, n_chunks)

    d0, d1 = agg(ones, zeros, srcp, dst2d)

    rb = functools.partial(_rows, ROW_BLOCK)
    sd = jax.ShapeDtypeStruct
    tca = _tc_call(
        _tca_body, (grid,),
        [rb(H), rb(H), rb(f_in), _full((f_in, H))],
        [rb(1), rb(H)],
        [sd((n_pad, 1), F32), sd((n_pad, H), F32)])
    dinv, y = tca(d0, d1, xts[0], W_g0)

    tc1 = _tc_call(
        _tc1_body, (grid,),
        [rb(H), rb(H), rb(f_in), rb(H), rb(H), rb(1), _full((1, H)),
         _full((f_in, 4 * H)), _full((H, 4 * H)), _full((H, 4 * H)),
         _full((1, 4 * H)), _full((H, H))],
        [rb(H), rb(H), rb(H)],
        [sd((n_pad, H), F32)] * 3)
    tc2 = _tc_call(
        _tc2_body, (grid,),
        [rb(H), rb(H), rb(H), rb(H), rb(H), rb(f_in), rb(1), _full((1, H)),
         _full((H, 4 * H)), _full((H, 4 * H)), _full((H, 4 * H)),
         _full((1, 4 * H)), _full((f_in, H))],
        [rb(H), rb(H), rb(H)],
        [sd((n_pad, H), F32)] * 3)

    h0 = c0 = h1 = c1 = zeros
    for t in range(t_steps):
        s0, s1 = agg(y, zeros, srcp, dst2d)
        h0, c0, y1 = tc1(s0, s1, xts[t], h0, c0, dinv, bg0,
                         wx0, wg0g, whh0t, bsum0, W_g1)
        s0, s1 = agg(y1, zeros, srcp, dst2d)
        h1, c1, y = tc2(s0, s1, h0, h1, c1, xts[(t + 1) % t_steps], dinv,
                        bg1, wh1, wg1g, whh1t, bsum1, W_g0)

    tcf = _tc_call(
        _tcf_body, (grid,),
        [rb(H), _full((H, W_out.shape[1])), _full((1, W_out.shape[1]))],
        rb(W_out.shape[1]),
        sd((n_pad, W_out.shape[1]), F32))
    out = tcf(h1, W_out, boutr)
    return out[:n]


# speedup vs baseline: 4.7452x; 1.1174x over previous
import functools

import jax
import jax.numpy as jnp
from jax import lax
from jax.experimental import pallas as pl
from jax.experimental.pallas import tpu as pltpu
from jax.experimental.pallas import tpu_sc as plsc

F32 = jnp.float32

N_CORES = 2
N_SUBCORES = 16
NW = N_CORES * N_SUBCORES
CHUNK = 128
ROW_BLOCK = 1024
H = 128


def _sigmoid(x):
    return 1.0 / (1.0 + jnp.exp(-x))


@functools.lru_cache(maxsize=None)
def _make_sc_agg(n_pad, n_chunks):
    epw = n_chunks * CHUNK
    rows_per_tile = n_pad // N_SUBCORES
    nbuf = 2
    nidx = 2 * nbuf
    out_sd = jax.ShapeDtypeStruct((n_pad, H), F32)
    mesh = plsc.VectorSubcoreMesh(core_axis_name="c", subcore_axis_name="s")

    @functools.partial(
        pl.kernel,
        mesh=mesh,
        out_type=(out_sd, out_sd),
        scratch_types=[
            pltpu.VMEM((nidx, CHUNK), jnp.int32),
            pltpu.VMEM((nidx, CHUNK), jnp.int32),
            pltpu.VMEM((nbuf, CHUNK, H), F32),
            pltpu.VMEM_SHARED((n_pad, H), F32),
        ] + [pltpu.SemaphoreType.DMA] * (nbuf + nidx),
    )
    def agg(y_hbm, zeros_hbm, src_hbm, dst_hbm, s0_hbm, s1_hbm,
            src_v, dst_v, rows_v, acc, *sems):
        rsem = sems[:nbuf]
        isem = sems[nbuf:]
        cid = lax.axis_index("c")
        sid = lax.axis_index("s")
        wid = sid * N_CORES + cid
        base_e = wid * epw
        rows = pl.ds(sid * rows_per_tile, rows_per_tile)

        @pl.when(cid == 0)
        def _():
            pltpu.sync_copy(y_hbm.at[rows], acc.at[rows])

        @pl.when(cid != 0)
        def _():
            pltpu.sync_copy(zeros_hbm.at[rows], acc.at[rows])

        plsc.subcore_barrier()

        def _idx_dmas(ci, s):
            e0 = base_e + ci * CHUNK
            return (
                pltpu.make_async_copy(src_hbm.at[pl.ds(e0, CHUNK)],
                                      src_v.at[s], isem[s]),
                pltpu.make_async_copy(dst_hbm.at[pl.ds(e0, CHUNK)],
                                      dst_v.at[s], isem[s]),
            )

        def _idx_fetch(ci, s):
            for d in _idx_dmas(ci, s):
                d.start()

        def _idx_wait(ci, s):
            for d in _idx_dmas(ci, s):
                d.wait()

        def _row_dma(ci, b, s):
            return pltpu.make_async_copy(y_hbm.at[src_v.at[s]],
                                         rows_v.at[b], rsem[b])

        for s in range(nidx):
            _idx_fetch(s, s)
        for b in range(nbuf):
            _idx_wait(b, b)
            _row_dma(b, b, b).start()

        def body(j, carry):
            for s in range(nidx):
                ci = j * nidx + s
                b = s % nbuf
                _row_dma(ci, b, s).wait()
                pltpu.sync_copy(rows_v.at[b], acc.at[dst_v.at[s]], add=True)

                @pl.when(ci + nidx < n_chunks)
                def _():
                    _idx_fetch(ci + nidx, s)

                @pl.when(ci + nbuf < n_chunks)
                def _():
                    s2 = (s + nbuf) % nidx
                    _idx_wait(ci + nbuf, s2)
                    _row_dma(ci + nbuf, b, s2).start()
            return carry

        lax.fori_loop(0, n_chunks // nidx, body, 0)
        plsc.subcore_barrier()

        @pl.when(cid == 0)
        def _():
            pltpu.sync_copy(acc.at[rows], s0_hbm.at[rows])

        @pl.when(cid != 0)
        def _():
            pltpu.sync_copy(acc.at[rows], s1_hbm.at[rows])

    return agg


def _tc_call(body, grid, in_specs, out_specs, out_shape):
    return pl.pallas_call(body, grid=grid, in_specs=in_specs,
                          out_specs=out_specs, out_shape=out_shape)


def _rows(b, c):
    return pl.BlockSpec((b, c), lambda i: (i, 0))


def _full(shape):
    return pl.BlockSpec(shape, lambda i: tuple(0 for _ in shape))


def _tca_body(d0, d1, x0, wg0, dinv_o, y0_o):
    dinv = lax.rsqrt(d0[:, :1] + d1[:, :1])
    dinv_o[...] = dinv
    y0_o[...] = dinv * jnp.dot(x0[...], wg0[...], preferred_element_type=F32)


def _lstm_step(s0, s1, dinv, bg, inp_a, wa, wg, h, whh, bsum, c):
    g = _sigmoid(dinv * (s0 + s1) + bg)
    gates = (jnp.dot(inp_a, wa, preferred_element_type=F32)
             + jnp.dot(g, wg, preferred_element_type=F32)
             + jnp.dot(h, whh, preferred_element_type=F32) + bsum)
    ig = _sigmoid(gates[:, 0:H])
    fg = _sigmoid(gates[:, H:2 * H])
    gg = jnp.tanh(gates[:, 2 * H:3 * H])
    og = _sigmoid(gates[:, 3 * H:4 * H])
    c2 = fg * c + ig * gg
    h2 = og * jnp.tanh(c2)
    return h2, c2


def _tc1_body(s0, s1, xt, h0, c0, dinv, bg0, wx, wgg, whh, bsum, wg1,
              h0n, c0n, y1n):
    dv = dinv[...]
    h2, c2 = _lstm_step(s0[...], s1[...], dv, bg0[...], xt[...], wx[...],
                        wgg[...], h0[...], whh[...], bsum[...], c0[...])
    h0n[...] = h2
    c0n[...] = c2
    y1n[...] = dv * jnp.dot(h2, wg1[...], preferred_element_type=F32)


def _tc2_body(s0, s1, h0, h1, c1, xn, dinv, bg1, wh, wgg, whh, bsum, wg0,
              h1n, c1n, y0n):
    dv = dinv[...]
    h2, c2 = _lstm_step(s0[...], s1[...], dv, bg1[...], h0[...], wh[...],
                        wgg[...], h1[...], whh[...], bsum[...], c1[...])
    h1n[...] = h2
    c1n[...] = c2
    y0n[...] = dv * jnp.dot(xn[...], wg0[...], preferred_element_type=F32)


def _tcf_body(h1, wout, bout, out):
    out[...] = jnp.dot(h1[...], wout[...], preferred_element_type=F32) + bout[...]


def kernel(x, edge_index, edge_attr,
           W_g0, b_g0, Wih0, Whh0, bih0, bhh0,
           W_g1, b_g1, Wih1, Whh1, bih1, bhh1,
           W_out, b_out):
    n, f_in, t_steps = x.shape
    n_pad = -(-n // ROW_BLOCK) * ROW_BLOCK
    e = edge_index.shape[1]
    n_chunks = -(-e // (NW * CHUNK))
    n_chunks = -(-n_chunks // 8) * 8
    e_pad = NW * n_chunks * CHUNK
    grid = n_pad // ROW_BLOCK

    src = edge_index[0].astype(jnp.int32)
    dst = edge_index[1].astype(jnp.int32)
    srcp = jnp.concatenate([src, jnp.zeros((e_pad - e,), jnp.int32)])
    dstp = jnp.concatenate([dst, jnp.full((e_pad - e,), n_pad - 1, jnp.int32)])

    zeros = jnp.zeros((n_pad, H), F32)
    ones = jnp.ones((n_pad, H), F32)
    xp = jnp.pad(x, ((0, n_pad - n), (0, 0), (0, 0)))
    xts = jnp.transpose(xp, (2, 0, 1))

    w0t = Wih0.T
    wx0, wg0g = w0t[:f_in], w0t[f_in:]
    whh0t = Whh0.T
    bsum0 = (bih0 + bhh0).reshape(1, 4 * H)
    w1t = Wih1.T
    wh1, wg1g = w1t[:H], w1t[H:]
    whh1t = Whh1.T
    bsum1 = (bih1 + bhh1).reshape(1, 4 * H)
    bg0 = b_g0.reshape(1, H)
    bg1 = b_g1.reshape(1, H)
    boutr = b_out.reshape(1, -1)

    agg = _make_sc_agg(n_pad, n_chunks)

    d0, d1 = agg(ones, zeros, srcp, dstp)

    rb = functools.partial(_rows, ROW_BLOCK)
    sd = jax.ShapeDtypeStruct
    tca = _tc_call(
        _tca_body, (grid,),
        [rb(H), rb(H), rb(f_in), _full((f_in, H))],
        [rb(1), rb(H)],
        [sd((n_pad, 1), F32), sd((n_pad, H), F32)])
    dinv, y = tca(d0, d1, xts[0], W_g0)

    tc1 = _tc_call(
        _tc1_body, (grid,),
        [rb(H), rb(H), rb(f_in), rb(H), rb(H), rb(1), _full((1, H)),
         _full((f_in, 4 * H)), _full((H, 4 * H)), _full((H, 4 * H)),
         _full((1, 4 * H)), _full((H, H))],
        [rb(H), rb(H), rb(H)],
        [sd((n_pad, H), F32)] * 3)
    tc2 = _tc_call(
        _tc2_body, (grid,),
        [rb(H), rb(H), rb(H), rb(H), rb(H), rb(f_in), rb(1), _full((1, H)),
         _full((H, 4 * H)), _full((H, 4 * H)), _full((H, 4 * H)),
         _full((1, 4 * H)), _full((f_in, H))],
        [rb(H), rb(H), rb(H)],
        [sd((n_pad, H), F32)] * 3)

    h0 = c0 = h1 = c1 = zeros
    for t in range(t_steps):
        s0, s1 = agg(y, zeros, srcp, dstp)
        h0, c0, y1 = tc1(s0, s1, xts[t], h0, c0, dinv, bg0,
                         wx0, wg0g, whh0t, bsum0, W_g1)
        s0, s1 = agg(y1, zeros, srcp, dstp)
        h1, c1, y = tc2(s0, s1, h0, h1, c1, xts[(t + 1) % t_steps], dinv,
                        bg1, wh1, wg1g, whh1t, bsum1, W_g0)

    tcf = _tc_call(
        _tcf_body, (grid,),
        [rb(H), _full((H, W_out.shape[1])), _full((1, W_out.shape[1]))],
        rb(W_out.shape[1]),
        sd((n_pad, W_out.shape[1]), F32))
    out = tcf(h1, W_out, boutr)
    return out[:n]


# speedup vs baseline: 4.7475x; 1.0005x over previous
import functools

import jax
import jax.numpy as jnp
from jax import lax
from jax.experimental import pallas as pl
from jax.experimental.pallas import tpu as pltpu
from jax.experimental.pallas import tpu_sc as plsc

F32 = jnp.float32

N_CORES = 2
N_SUBCORES = 16
NW = N_CORES * N_SUBCORES
CHUNK = 64
NBUF = 4
NIDX = 2 * NBUF
ROW_BLOCK = 1024
H = 128


def _sigmoid(x):
    return 1.0 / (1.0 + jnp.exp(-x))


@functools.lru_cache(maxsize=None)
def _make_sc_agg(n_pad, n_chunks):
    epw = n_chunks * CHUNK
    rows_per_tile = n_pad // N_SUBCORES
    out_sd = jax.ShapeDtypeStruct((n_pad, H), F32)
    mesh = plsc.VectorSubcoreMesh(core_axis_name="c", subcore_axis_name="s")

    @functools.partial(
        pl.kernel,
        mesh=mesh,
        out_type=(out_sd, out_sd),
        scratch_types=[
            pltpu.VMEM((NIDX, CHUNK), jnp.int32),
            pltpu.VMEM((NIDX, CHUNK), jnp.int32),
            pltpu.VMEM((NBUF, CHUNK, H), F32),
            pltpu.VMEM_SHARED((n_pad, H), F32),
        ] + [pltpu.SemaphoreType.DMA] * (NBUF + NIDX),
    )
    def agg(y_hbm, zeros_hbm, src_hbm, dst_hbm, s0_hbm, s1_hbm,
            src_v, dst_v, rows_v, acc, *sems):
        rsem = sems[:NBUF]
        isem = sems[NBUF:]
        cid = lax.axis_index("c")
        sid = lax.axis_index("s")
        wid = sid * N_CORES + cid
        base_e = wid * epw
        rows = pl.ds(sid * rows_per_tile, rows_per_tile)

        @pl.when(cid == 0)
        def _():
            pltpu.sync_copy(y_hbm.at[rows], acc.at[rows])

        @pl.when(cid != 0)
        def _():
            pltpu.sync_copy(zeros_hbm.at[rows], acc.at[rows])

        plsc.subcore_barrier()

        def _idx_dmas(ci, s):
            e0 = base_e + ci * CHUNK
            return (
                pltpu.make_async_copy(src_hbm.at[pl.ds(e0, CHUNK)],
                                      src_v.at[s], isem[s]),
                pltpu.make_async_copy(dst_hbm.at[pl.ds(e0, CHUNK)],
                                      dst_v.at[s], isem[s]),
            )

        def _idx_fetch(ci, s):
            for d in _idx_dmas(ci, s):
                d.start()

        def _idx_wait(ci, s):
            for d in _idx_dmas(ci, s):
                d.wait()

        def _row_dma(b, s):
            return pltpu.make_async_copy(y_hbm.at[src_v.at[s]],
                                         rows_v.at[b], rsem[b])

        for s in range(NIDX):
            _idx_fetch(s, s)
        for b in range(NBUF):
            _idx_wait(b, b)
            _row_dma(b, b).start()

        def body(j, carry):
            for s in range(NIDX):
                ci = j * NIDX + s
                b = s % NBUF
                _row_dma(b, s).wait()
                pltpu.sync_copy(rows_v.at[b], acc.at[dst_v.at[s]], add=True)

                @pl.when(ci + NIDX < n_chunks)
                def _():
                    _idx_fetch(ci + NIDX, s)

                @pl.when(ci + NBUF < n_chunks)
                def _():
                    s2 = (s + NBUF) % NIDX
                    _idx_wait(ci + NBUF, s2)
                    _row_dma(b, s2).start()
            return carry

        lax.fori_loop(0, n_chunks // NIDX, body, 0)
        plsc.subcore_barrier()

        @pl.when(cid == 0)
        def _():
            pltpu.sync_copy(acc.at[rows], s0_hbm.at[rows])

        @pl.when(cid != 0)
        def _():
            pltpu.sync_copy(acc.at[rows], s1_hbm.at[rows])

    return agg


def _tc_call(body, grid, in_specs, out_specs, out_shape):
    return pl.pallas_call(body, grid=grid, in_specs=in_specs,
                          out_specs=out_specs, out_shape=out_shape)


def _rows(b, c):
    return pl.BlockSpec((b, c), lambda i: (i, 0))


def _full(shape):
    return pl.BlockSpec(shape, lambda i: tuple(0 for _ in shape))


def _tca_body(d0, d1, x0, wg0, dinv_o, y0_o):
    dinv = lax.rsqrt(d0[:, :1] + d1[:, :1])
    dinv_o[...] = dinv
    y0_o[...] = dinv * jnp.dot(x0[...], wg0[...], preferred_element_type=F32)


def _lstm_step(s0, s1, dinv, bg, inp_a, wa, wg, h, whh, bsum, c):
    g = _sigmoid(dinv * (s0 + s1) + bg)
    gates = (jnp.dot(inp_a, wa, preferred_element_type=F32)
             + jnp.dot(g, wg, preferred_element_type=F32)
             + jnp.dot(h, whh, preferred_element_type=F32) + bsum)
    ig = _sigmoid(gates[:, 0:H])
    fg = _sigmoid(gates[:, H:2 * H])
    gg = jnp.tanh(gates[:, 2 * H:3 * H])
    og = _sigmoid(gates[:, 3 * H:4 * H])
    c2 = fg * c + ig * gg
    h2 = og * jnp.tanh(c2)
    return h2, c2


def _tc1_body(s0, s1, xt, h0, c0, dinv, bg0, wx, wgg, whh, bsum, wg1,
              h0n, c0n, y1n):
    dv = dinv[...]
    h2, c2 = _lstm_step(s0[...], s1[...], dv, bg0[...], xt[...], wx[...],
                        wgg[...], h0[...], whh[...], bsum[...], c0[...])
    h0n[...] = h2
    c0n[...] = c2
    y1n[...] = dv * jnp.dot(h2, wg1[...], preferred_element_type=F32)


def _tc2_body(s0, s1, h0, h1, c1, xn, dinv, bg1, wh, wgg, whh, bsum, wg0,
              h1n, c1n, y0n):
    dv = dinv[...]
    h2, c2 = _lstm_step(s0[...], s1[...], dv, bg1[...], h0[...], wh[...],
                        wgg[...], h1[...], whh[...], bsum[...], c1[...])
    h1n[...] = h2
    c1n[...] = c2
    y0n[...] = dv * jnp.dot(xn[...], wg0[...], preferred_element_type=F32)


def _tcf_body(h1, wout, bout, out):
    out[...] = jnp.dot(h1[...], wout[...], preferred_element_type=F32) + bout[...]


def kernel(x, edge_index, edge_attr,
           W_g0, b_g0, Wih0, Whh0, bih0, bhh0,
           W_g1, b_g1, Wih1, Whh1, bih1, bhh1,
           W_out, b_out):
    n, f_in, t_steps = x.shape
    n_pad = -(-n // ROW_BLOCK) * ROW_BLOCK
    e = edge_index.shape[1]
    n_chunks = -(-e // (NW * CHUNK))
    n_chunks = -(-n_chunks // NIDX) * NIDX
    e_pad = NW * n_chunks * CHUNK
    grid = n_pad // ROW_BLOCK

    src = edge_index[0].astype(jnp.int32)
    dst = edge_index[1].astype(jnp.int32)
    srcp = jnp.concatenate([src, jnp.zeros((e_pad - e,), jnp.int32)])
    dstp = jnp.concatenate([dst, jnp.full((e_pad - e,), n_pad - 1, jnp.int32)])

    zeros = jnp.zeros((n_pad, H), F32)
    ones = jnp.ones((n_pad, H), F32)
    xp = jnp.pad(x, ((0, n_pad - n), (0, 0), (0, 0)))
    xts = jnp.transpose(xp, (2, 0, 1))

    w0t = Wih0.T
    wx0, wg0g = w0t[:f_in], w0t[f_in:]
    whh0t = Whh0.T
    bsum0 = (bih0 + bhh0).reshape(1, 4 * H)
    w1t = Wih1.T
    wh1, wg1g = w1t[:H], w1t[H:]
    whh1t = Whh1.T
    bsum1 = (bih1 + bhh1).reshape(1, 4 * H)
    bg0 = b_g0.reshape(1, H)
    bg1 = b_g1.reshape(1, H)
    boutr = b_out.reshape(1, -1)

    agg = _make_sc_agg(n_pad, n_chunks)

    d0, d1 = agg(ones, zeros, srcp, dstp)

    rb = functools.partial(_rows, ROW_BLOCK)
    sd = jax.ShapeDtypeStruct
    tca = _tc_call(
        _tca_body, (grid,),
        [rb(H), rb(H), rb(f_in), _full((f_in, H))],
        [rb(1), rb(H)],
        [sd((n_pad, 1), F32), sd((n_pad, H), F32)])
    dinv, y = tca(d0, d1, xts[0], W_g0)

    tc1 = _tc_call(
        _tc1_body, (grid,),
        [rb(H), rb(H), rb(f_in), rb(H), rb(H), rb(1), _full((1, H)),
         _full((f_in, 4 * H)), _full((H, 4 * H)), _full((H, 4 * H)),
         _full((1, 4 * H)), _full((H, H))],
        [rb(H), rb(H), rb(H)],
        [sd((n_pad, H), F32)] * 3)
    tc2 = _tc_call(
        _tc2_body, (grid,),
        [rb(H), rb(H), rb(H), rb(H), rb(H), rb(f_in), rb(1), _full((1, H)),
         _full((H, 4 * H)), _full((H, 4 * H)), _full((H, 4 * H)),
         _full((1, 4 * H)), _full((f_in, H))],
        [rb(H), rb(H), rb(H)],
        [sd((n_pad, H), F32)] * 3)

    h0 = c0 = h1 = c1 = zeros
    for t in range(t_steps):
        s0, s1 = agg(y, zeros, srcp, dstp)
        h0, c0, y1 = tc1(s0, s1, xts[t], h0, c0, dinv, bg0,
                         wx0, wg0g, whh0t, bsum0, W_g1)
        s0, s1 = agg(y1, zeros, srcp, dstp)
        h1, c1, y = tc2(s0, s1, h0, h1, c1, xts[(t + 1) % t_steps], dinv,
                        bg1, wh1, wg1g, whh1t, bsum1, W_g0)

    tcf = _tc_call(
        _tcf_body, (grid,),
        [rb(H), _full((H, W_out.shape[1])), _full((1, W_out.shape[1]))],
        rb(W_out.shape[1]),
        sd((n_pad, W_out.shape[1]), F32))
    out = tcf(h1, W_out, boutr)
    return out[:n]
